# Initial kernel scaffold; baseline (speedup 1.0000x reference)
import jax
import jax.numpy as jnp
from jax.experimental import pallas as pl


def kernel(x, edge_index, W1, b1, W2, b2, Wfc, bfc):
    raise NotImplementedError("write your pallas kernel here")


# speedup vs baseline: 20.4507x; 20.4507x over previous
import functools

import jax
import jax.numpy as jnp
from jax import lax
from jax.experimental import pallas as pl
from jax.experimental.pallas import tpu as pltpu
from jax.experimental.pallas import tpu_sc as plsc

N_NODES = 10000
N_EDGES = 320000
NFEAT = 128

NC = 2
NS = 16
NW = NC * NS
EPT = N_EDGES // NW
CHUNK = 200
NITER = EPT // CHUNK
ROWS_PER_TILE = 640

_mesh = plsc.VectorSubcoreMesh(core_axis_name="c", subcore_axis_name="s")


@functools.partial(
    pl.kernel,
    mesh=_mesh,
    out_type=jax.ShapeDtypeStruct((NW, N_NODES), jnp.float32),
    scratch_types=[
        pltpu.VMEM((N_NODES,), jnp.float32),
        pltpu.VMEM((EPT,), jnp.int32),
    ],
    compiler_params=pltpu.CompilerParams(needs_layout_passes=False),
)
def _deg_kernel(dst_hbm, out_hbm, hist, dstv):
    c = lax.axis_index("c")
    s = lax.axis_index("s")
    wid = c * NS + s

    def zero(i, carry):
        hist[pl.ds(i * 16, 16)] = jnp.zeros((16,), jnp.float32)
        return carry

    lax.fori_loop(0, N_NODES // 16, zero, 0)

    pltpu.sync_copy(dst_hbm.at[pl.ds(wid * EPT, EPT)], dstv)
    ones = jnp.ones((16,), jnp.float32)

    def step(i, carry):
        idx = dstv[pl.ds(i * 16, 16)]
        plsc.addupdate_scatter(hist, [idx], ones)
        return carry

    lax.fori_loop(0, EPT // 16, step, 0)
    pltpu.sync_copy(hist, out_hbm.at[wid])


@functools.partial(
    pl.kernel,
    mesh=_mesh,
    out_type=jax.ShapeDtypeStruct((NC, N_NODES, NFEAT), jnp.float32),
    scratch_types=[
        pltpu.VMEM_SHARED((N_NODES, NFEAT), jnp.float32),
        pltpu.VMEM((CHUNK,), jnp.int32),
        pltpu.VMEM((CHUNK,), jnp.int32),
        pltpu.VMEM((CHUNK, NFEAT), jnp.float32),
        pltpu.SemaphoreType.DMA,
    ],
    compiler_params=pltpu.CompilerParams(needs_layout_passes=False),
)
def _scatter_kernel(h_hbm, src_hbm, dst_hbm, zeros_hbm, out_hbm,
                    acc, srcv, dstv, rows, sem):
    c = lax.axis_index("c")
    s = lax.axis_index("s")

    @pl.when(s < NS - 1)
    def _():
        pltpu.sync_copy(zeros_hbm.at[pl.ds(s * ROWS_PER_TILE, ROWS_PER_TILE)],
                        acc.at[pl.ds(s * ROWS_PER_TILE, ROWS_PER_TILE)])

    @pl.when(s == NS - 1)
    def _():
        last = (NS - 1) * ROWS_PER_TILE
        pltpu.sync_copy(zeros_hbm.at[pl.ds(last, N_NODES - last)],
                        acc.at[pl.ds(last, N_NODES - last)])

    plsc.subcore_barrier()

    wid = c * NS + s
    ebase = wid * EPT

    def step(i, carry):
        base = ebase + i * CHUNK
        pltpu.sync_copy(src_hbm.at[pl.ds(base, CHUNK)], srcv)
        pltpu.sync_copy(dst_hbm.at[pl.ds(base, CHUNK)], dstv)
        pltpu.async_copy(h_hbm.at[srcv], rows, sem).wait()
        pltpu.sync_copy(rows, acc.at[dstv], add=True)
        return carry

    lax.fori_loop(0, NITER, step, 0)
    plsc.subcore_barrier()

    @pl.when(s < NS - 1)
    def _():
        pltpu.sync_copy(acc.at[pl.ds(s * ROWS_PER_TILE, ROWS_PER_TILE)],
                        out_hbm.at[c, pl.ds(s * ROWS_PER_TILE, ROWS_PER_TILE)])

    @pl.when(s == NS - 1)
    def _():
        last = (NS - 1) * ROWS_PER_TILE
        pltpu.sync_copy(acc.at[pl.ds(last, N_NODES - last)],
                        out_hbm.at[c, pl.ds(last, N_NODES - last)])


_RB = 1000
_GRID = N_NODES // _RB


def _mm1_body(x_ref, w_ref, o_ref):
    o_ref[...] = jnp.dot(x_ref[...], w_ref[...],
                         preferred_element_type=jnp.float32)


def _mm1(x, w):
    return pl.pallas_call(
        _mm1_body,
        grid=(_GRID,),
        in_specs=[
            pl.BlockSpec((_RB, NFEAT), lambda i: (i, 0)),
            pl.BlockSpec((NFEAT, NFEAT), lambda i: (0, 0)),
        ],
        out_specs=pl.BlockSpec((_RB, NFEAT), lambda i: (i, 0)),
        out_shape=jax.ShapeDtypeStruct((N_NODES, NFEAT), jnp.float32),
    )(x, w)


def _dinv_body(hist_ref, dinv_ref):
    deg = 1.0 + jnp.sum(hist_ref[...], axis=0)
    dinv_ref[...] = lax.rsqrt(deg)[:, None]


def _dinv(hist):
    return pl.pallas_call(
        _dinv_body,
        grid=(1,),
        in_specs=[pl.BlockSpec((NW, N_NODES), lambda i: (0, 0))],
        out_specs=pl.BlockSpec((N_NODES, 1), lambda i: (0, 0)),
        out_shape=jax.ShapeDtypeStruct((N_NODES, 1), jnp.float32),
    )(hist)


def _scale_body(h1_ref, dinv_ref, h1p_ref):
    h1p_ref[...] = h1_ref[...] * dinv_ref[...]


def _scale(h1, dinv):
    return pl.pallas_call(
        _scale_body,
        grid=(_GRID,),
        in_specs=[
            pl.BlockSpec((_RB, NFEAT), lambda i: (i, 0)),
            pl.BlockSpec((_RB, 1), lambda i: (i, 0)),
        ],
        out_specs=pl.BlockSpec((_RB, NFEAT), lambda i: (i, 0)),
        out_shape=jax.ShapeDtypeStruct((N_NODES, NFEAT), jnp.float32),
    )(h1, dinv)


def _mid_body(p_ref, h1p_ref, dinv_ref, b1_ref, w2_ref, h2p_ref):
    psum = p_ref[0] + p_ref[1]
    u = (psum + h1p_ref[...]) * dinv_ref[...] + b1_ref[...]
    u = jnp.maximum(u, 0.0)
    h2 = jnp.dot(u, w2_ref[...], preferred_element_type=jnp.float32)
    h2p_ref[...] = h2 * dinv_ref[...]


def _mid(p1, h1p, dinv, b1, w2):
    return pl.pallas_call(
        _mid_body,
        grid=(_GRID,),
        in_specs=[
            pl.BlockSpec((NC, _RB, NFEAT), lambda i: (0, i, 0)),
            pl.BlockSpec((_RB, NFEAT), lambda i: (i, 0)),
            pl.BlockSpec((_RB, 1), lambda i: (i, 0)),
            pl.BlockSpec((1, NFEAT), lambda i: (0, 0)),
            pl.BlockSpec((NFEAT, NFEAT), lambda i: (0, 0)),
        ],
        out_specs=pl.BlockSpec((_RB, NFEAT), lambda i: (i, 0)),
        out_shape=jax.ShapeDtypeStruct((N_NODES, NFEAT), jnp.float32),
    )(p1, h1p, dinv, b1, w2)


def _fin_body(p_ref, h2p_ref, dinv_ref, b2_ref, wfc_ref, bfc_ref, o_ref):
    v = (p_ref[0] + p_ref[1] + h2p_ref[...]) * dinv_ref[...] + b2_ref[...]
    o_ref[...] = jnp.dot(v, wfc_ref[...],
                         preferred_element_type=jnp.float32) + bfc_ref[0, 0]


def _fin(p2, h2p, dinv, b2, wfc, bfc):
    return pl.pallas_call(
        _fin_body,
        grid=(_GRID,),
        in_specs=[
            pl.BlockSpec((NC, _RB, NFEAT), lambda i: (0, i, 0)),
            pl.BlockSpec((_RB, NFEAT), lambda i: (i, 0)),
            pl.BlockSpec((_RB, 1), lambda i: (i, 0)),
            pl.BlockSpec((1, NFEAT), lambda i: (0, 0)),
            pl.BlockSpec((NFEAT, 1), lambda i: (0, 0)),
            pl.BlockSpec((1, 1), lambda i: (0, 0)),
        ],
        out_specs=pl.BlockSpec((_RB, 1), lambda i: (i, 0)),
        out_shape=jax.ShapeDtypeStruct((N_NODES, 1), jnp.float32),
    )(p2, h2p, dinv, b2, wfc, bfc)


def kernel(x, edge_index, W1, b1, W2, b2, Wfc, bfc):
    ei = edge_index.astype(jnp.int32)
    src = ei[0]
    dst = ei[1]
    zeros = jnp.zeros((N_NODES, NFEAT), jnp.float32)

    hist = _deg_kernel(dst)
    h1 = _mm1(x, W1)
    dinv = _dinv(hist)
    h1p = _scale(h1, dinv)
    p1 = _scatter_kernel(h1p, src, dst, zeros)
    h2p = _mid(p1, h1p, dinv, b1.reshape(1, NFEAT), W2)
    p2 = _scatter_kernel(h2p, src, dst, zeros)
    out = _fin(p2, h2p, dinv, b2.reshape(1, NFEAT), Wfc, bfc.reshape(1, 1))
    return out.reshape(N_NODES)
